# baseline (speedup 1.0000x reference)
import jax
import jax.numpy as jnp
from jax import lax
from jax.experimental import pallas as pl

T = 1024
D = 1024
E = 8
FF = 512


def _combine_all(x, wr):
    logits = lax.dot_general(x, wr, (((1,), (1,)), ((), ())),
                             preferred_element_type=jnp.float32)
    probs = jax.nn.softmax(logits, axis=-1)
    col = lax.broadcasted_iota(jnp.int32, probs.shape, 1)
    v1 = jnp.max(probs, axis=-1, keepdims=True)
    i1 = jnp.min(jnp.where(probs == v1, col, E), axis=-1, keepdims=True)
    masked = jnp.where(col == i1, -jnp.inf, probs)
    v2 = jnp.max(masked, axis=-1, keepdims=True)
    i2 = jnp.min(jnp.where(masked == v2, col, E), axis=-1, keepdims=True)
    s = v1 + v2
    w1 = v1 / s
    w2 = v2 / s
    return jnp.where(col == i1, w1, 0.0) + jnp.where(col == i2, w2, 0.0)


def _moe_body(x_ref, wr_ref, wg_ref, wu_ref, wd0_ref, wd1_ref, out_ref,
              c_ref, xb_ref):
    e = pl.program_id(0)

    @pl.when(e == 0)
    def _():
        x = x_ref[...]
        c_ref[...] = _combine_all(x, wr_ref[...])
        xb_ref[...] = x.astype(jnp.bfloat16)

    col = lax.broadcasted_iota(jnp.int32, (T, E), 1)
    c_e = jnp.sum(jnp.where(col == e, c_ref[...], 0.0), axis=1, keepdims=True)
    xb = xb_ref[...]
    gate = lax.dot_general(xb, wg_ref[0].astype(jnp.bfloat16),
                           (((1,), (1,)), ((), ())),
                           preferred_element_type=jnp.float32)
    up = lax.dot_general(xb, wu_ref[0].astype(jnp.bfloat16),
                         (((1,), (1,)), ((), ())),
                         preferred_element_type=jnp.float32)
    act = ((gate * jax.nn.sigmoid(gate) * up) * c_e).astype(jnp.bfloat16)
    wd = jnp.concatenate([wd0_ref[0], wd1_ref[0]], axis=0)
    contrib = lax.dot_general(act, wd.astype(jnp.bfloat16),
                              (((1,), (1,)), ((), ())),
                              preferred_element_type=jnp.float32)

    @pl.when(e == 0)
    def _():
        out_ref[...] = contrib

    @pl.when(e != 0)
    def _():
        out_ref[...] = out_ref[...] + contrib


def kernel(hidden_states, router_weight, w_gate_up, w_down):
    from jax.experimental.pallas import tpu as pltpu
    return pl.pallas_call(
        _moe_body,
        grid=(E,),
        in_specs=[
            pl.BlockSpec((T, D), lambda e: (0, 0)),
            pl.BlockSpec((E, D), lambda e: (0, 0)),
            pl.BlockSpec((1, FF, D), lambda e: (e, 0, 0)),
            pl.BlockSpec((1, FF, D), lambda e: (e, 1, 0)),
            pl.BlockSpec((1, D // 2, FF), lambda e: (e, 0, 0)),
            pl.BlockSpec((1, D // 2, FF), lambda e: (e, 1, 0)),
        ],
        out_specs=pl.BlockSpec((T, D), lambda e: (0, 0)),
        out_shape=jax.ShapeDtypeStruct((T, D), jnp.float32),
        scratch_shapes=[pltpu.VMEM((T, E), jnp.float32),
                        pltpu.VMEM((T, D), jnp.bfloat16)],
    )(hidden_states, router_weight, w_gate_up, w_gate_up, w_down, w_down)


# speedup vs baseline: 1.0030x; 1.0030x over previous
import jax
import jax.numpy as jnp
from jax import lax
from jax.experimental import pallas as pl

T = 1024
D = 1024
E = 8
FF = 512
EPS = 2


def _combine_all(x, wr):
    logits = lax.dot_general(x, wr, (((1,), (1,)), ((), ())),
                             preferred_element_type=jnp.float32)
    probs = jax.nn.softmax(logits, axis=-1)
    col = lax.broadcasted_iota(jnp.int32, probs.shape, 1)
    v1 = jnp.max(probs, axis=-1, keepdims=True)
    i1 = jnp.min(jnp.where(probs == v1, col, E), axis=-1, keepdims=True)
    masked = jnp.where(col == i1, -jnp.inf, probs)
    v2 = jnp.max(masked, axis=-1, keepdims=True)
    i2 = jnp.min(jnp.where(masked == v2, col, E), axis=-1, keepdims=True)
    s = v1 + v2
    w1 = v1 / s
    w2 = v2 / s
    return jnp.where(col == i1, w1, 0.0) + jnp.where(col == i2, w2, 0.0)


def _expert_contrib(xb, wgu, wd, c_e):
    gu = lax.dot_general(xb, wgu.astype(jnp.bfloat16),
                         (((1,), (1,)), ((), ())),
                         preferred_element_type=jnp.float32)
    gate = gu[:, :FF]
    up = gu[:, FF:]
    act = ((gate * jax.nn.sigmoid(gate) * up) * c_e).astype(jnp.bfloat16)
    return lax.dot_general(act, wd.astype(jnp.bfloat16),
                           (((1,), (1,)), ((), ())),
                           preferred_element_type=jnp.float32)


def _moe_body(x_ref, wr_ref, wgu_ref, wd_ref, out_ref, c_ref, xb_ref):
    g = pl.program_id(0)

    @pl.when(g == 0)
    def _():
        x = x_ref[...]
        c_ref[...] = _combine_all(x, wr_ref[...])
        xb_ref[...] = x.astype(jnp.bfloat16)

    col = lax.broadcasted_iota(jnp.int32, (T, E), 1)
    xb = xb_ref[...]
    c = c_ref[...]
    c0 = jnp.sum(jnp.where(col == EPS * g, c, 0.0), axis=1, keepdims=True)
    c1 = jnp.sum(jnp.where(col == EPS * g + 1, c, 0.0), axis=1, keepdims=True)
    contrib = (_expert_contrib(xb, wgu_ref[0], wd_ref[0], c0) +
               _expert_contrib(xb, wgu_ref[1], wd_ref[1], c1))

    @pl.when(g == 0)
    def _():
        out_ref[...] = contrib

    @pl.when(g != 0)
    def _():
        out_ref[...] = out_ref[...] + contrib


def kernel(hidden_states, router_weight, w_gate_up, w_down):
    from jax.experimental.pallas import tpu as pltpu
    return pl.pallas_call(
        _moe_body,
        grid=(E // EPS,),
        in_specs=[
            pl.BlockSpec((T, D), lambda g: (0, 0)),
            pl.BlockSpec((E, D), lambda g: (0, 0)),
            pl.BlockSpec((EPS, 2 * FF, D), lambda g: (g, 0, 0)),
            pl.BlockSpec((EPS, D, FF), lambda g: (g, 0, 0)),
        ],
        out_specs=pl.BlockSpec((T, D), lambda g: (0, 0)),
        out_shape=jax.ShapeDtypeStruct((T, D), jnp.float32),
        scratch_shapes=[pltpu.VMEM((T, E), jnp.float32),
                        pltpu.VMEM((T, D), jnp.bfloat16)],
    )(hidden_states, router_weight, w_gate_up, w_down)


# speedup vs baseline: 1.0090x; 1.0060x over previous
import jax
import jax.numpy as jnp
from jax import lax
from jax.experimental import pallas as pl

T = 1024
D = 1024
E = 8
FF = 512


def _combine_all(x, wr):
    logits = lax.dot_general(x, wr, (((1,), (1,)), ((), ())),
                             preferred_element_type=jnp.float32)
    probs = jax.nn.softmax(logits, axis=-1)
    col = lax.broadcasted_iota(jnp.int32, probs.shape, 1)
    v1 = jnp.max(probs, axis=-1, keepdims=True)
    i1 = jnp.min(jnp.where(probs == v1, col, E), axis=-1, keepdims=True)
    masked = jnp.where(col == i1, -jnp.inf, probs)
    v2 = jnp.max(masked, axis=-1, keepdims=True)
    i2 = jnp.min(jnp.where(masked == v2, col, E), axis=-1, keepdims=True)
    s = v1 + v2
    w1 = v1 / s
    w2 = v2 / s
    return jnp.where(col == i1, w1, 0.0) + jnp.where(col == i2, w2, 0.0)


def _moe_body(x_ref, wr_ref, wgu_ref, wd_ref, out_ref, c_ref, xb_ref):
    e = pl.program_id(0)

    @pl.when(e == 0)
    def _():
        x = x_ref[...]
        c_ref[...] = _combine_all(x, wr_ref[...])
        xb_ref[...] = x.astype(jnp.bfloat16)

    col = lax.broadcasted_iota(jnp.int32, (T, E), 1)
    c_e = jnp.sum(jnp.where(col == e, c_ref[...], 0.0), axis=1, keepdims=True)
    xb = xb_ref[...]
    wgu = wgu_ref[0].astype(jnp.bfloat16)
    gu = lax.dot_general(xb, wgu, (((1,), (1,)), ((), ())),
                         preferred_element_type=jnp.float32)
    gate = gu[:, :FF]
    up = gu[:, FF:]
    act = (gate * jax.nn.sigmoid(gate) * up) * c_e
    wd = wd_ref[0].astype(jnp.bfloat16)
    contrib = lax.dot_general(act.astype(jnp.bfloat16), wd,
                              (((1,), (1,)), ((), ())),
                              preferred_element_type=jnp.float32)

    @pl.when(e == 0)
    def _():
        out_ref[...] = contrib

    @pl.when(e != 0)
    def _():
        out_ref[...] = out_ref[...] + contrib


def kernel(hidden_states, router_weight, w_gate_up, w_down):
    from jax.experimental.pallas import tpu as pltpu
    return pl.pallas_call(
        _moe_body,
        grid=(E,),
        in_specs=[
            pl.BlockSpec((T, D), lambda e: (0, 0)),
            pl.BlockSpec((E, D), lambda e: (0, 0)),
            pl.BlockSpec((1, 2 * FF, D), lambda e: (e, 0, 0)),
            pl.BlockSpec((1, D, FF), lambda e: (e, 0, 0)),
        ],
        out_specs=pl.BlockSpec((T, D), lambda e: (0, 0)),
        out_shape=jax.ShapeDtypeStruct((T, D), jnp.float32),
        scratch_shapes=[pltpu.VMEM((T, E), jnp.float32),
                        pltpu.VMEM((T, D), jnp.bfloat16)],
    )(hidden_states, router_weight, w_gate_up, w_down)
